# baseline (speedup 1.0000x reference)
import functools
import math

import jax
import jax.numpy as jnp
from jax import lax
from jax.experimental import pallas as pl
from jax.experimental.pallas import tpu as pltpu
from jax.experimental.pallas import tpu_sc as plsc

_ALPHA, _BETA, _GAMMA = 0.1, 0.45, 0.45
_BLK = 512
_CH = 512
_NEG = -1e30


def _eps_values(blur=0.05, p=2, scaling=0.5, eps0=1.0):
    tgt = blur ** p
    out = []
    e = eps0
    while e > tgt:
        out.append(e)
        e *= scaling
    out += [tgt] * 5
    return out


_EPS = _eps_values()


def _sinkhorn_body(eps_ref, x_ref, yt_ref, of_ref, og_ref,
                   x2_ref, y2_ref, f_ref, g_ref, u_ref, v_ref):
    n = x_ref.shape[1]
    nb = n // _CH
    log_w = -math.log(n)

    x2_ref[...] = jnp.sum(x_ref[0] * x_ref[0], axis=1, keepdims=True)
    y2_ref[...] = jnp.sum(yt_ref[0] * yt_ref[0], axis=0, keepdims=True)
    f_ref[...] = jnp.zeros_like(f_ref)
    g_ref[...] = jnp.zeros_like(g_ref)

    ones_c = jnp.ones((n, 1), jnp.float32)
    zeros_c = jnp.zeros((n, 2), jnp.float32)
    ones_r = jnp.ones((1, n), jnp.float32)
    zeros_r = jnp.zeros((2, n), jnp.float32)

    log2e = 1.4426950408889634
    ln2 = 0.6931471805599453
    ones_1r = jnp.ones((1, _CH), jnp.float32)
    ones_1c = jnp.ones((_CH, 1), jnp.float32)

    def eps_step(t, _):
        eps = eps_ref[t]
        ie = log2e / eps

        u_ref[...] = jnp.concatenate(
            [x_ref[0][:, 0:4] * ie,
             (f_ref[...] - 0.5 * x2_ref[...]) * ie, ones_c, zeros_c], axis=1)
        v_ref[...] = jnp.concatenate(
            [yt_ref[0][0:4, :], ones_r, (-0.5 * ie) * y2_ref[...], zeros_r],
            axis=0)

        def g_block(jb, _):
            j0 = jb * _BLK
            vb = v_ref[:, pl.ds(j0, _BLK)]

            def chunk(ic, carry):
                m, s = carry
                uc = u_ref[pl.ds(ic * _CH, _CH), :]
                a = lax.dot_general(uc, vb, (((1,), (0,)), ((), ())),
                                    preferred_element_type=jnp.float32)
                mn = jnp.maximum(m, jnp.max(a, axis=0, keepdims=True))
                e = jnp.exp2(a - mn)
                es = lax.dot_general(ones_1r, e, (((1,), (0,)), ((), ())),
                                     preferred_element_type=jnp.float32)
                s = s * jnp.exp2(m - mn) + es
                return mn, s

            carry = (jnp.full((1, _BLK), _NEG, jnp.float32),
                     jnp.zeros((1, _BLK), jnp.float32))
            for ic in range(nb):
                carry = chunk(ic, carry)
            m, s = carry
            g_ref[:, pl.ds(j0, _BLK)] = -eps * (
                (jnp.log2(s) + m) * ln2 + log_w)
            return 0

        lax.fori_loop(0, nb, g_block, 0)

        u_ref[...] = jnp.concatenate(
            [x_ref[0][:, 0:4], ones_c, (-0.5 * ie) * x2_ref[...], zeros_c],
            axis=1)
        v_ref[...] = jnp.concatenate(
            [yt_ref[0][0:4, :] * ie,
             (g_ref[...] - 0.5 * y2_ref[...]) * ie, ones_r, zeros_r], axis=0)

        def f_block(ib, _):
            i0 = ib * _BLK
            ub = u_ref[pl.ds(i0, _BLK), :]

            def chunk(jc, carry):
                m, s = carry
                vc = v_ref[:, pl.ds(jc * _CH, _CH)]
                a = lax.dot_general(ub, vc, (((1,), (0,)), ((), ())),
                                    preferred_element_type=jnp.float32)
                mn = jnp.maximum(m, jnp.max(a, axis=1, keepdims=True))
                e = jnp.exp2(a - mn)
                es = lax.dot_general(e, ones_1c, (((1,), (0,)), ((), ())),
                                     preferred_element_type=jnp.float32)
                s = s * jnp.exp2(m - mn) + es
                return mn, s

            carry = (jnp.full((_BLK, 1), _NEG, jnp.float32),
                     jnp.zeros((_BLK, 1), jnp.float32))
            for jc in range(nb):
                carry = chunk(jc, carry)
            m, s = carry
            f_ref[pl.ds(i0, _BLK), :] = -eps * (
                (jnp.log2(s) + m) * ln2 + log_w)
            return 0

        lax.fori_loop(0, nb, f_block, 0)
        return 0

    lax.fori_loop(0, len(_EPS), eps_step, 0)
    fm = jnp.sum(f_ref[...]) / n
    gm = jnp.sum(g_ref[...]) / n
    of_ref[...] = fm * jnp.ones((1, 1, 128), jnp.float32)
    og_ref[...] = gm * jnp.ones((1, 1, 128), jnp.float32)


def _sinkhorn_pairs(xs, yts, eps_arr):
    b, n, d = xs.shape
    return pl.pallas_call(
        _sinkhorn_body,
        grid=(b,),
        in_specs=[
            pl.BlockSpec(memory_space=pltpu.SMEM),
            pl.BlockSpec((1, n, d), lambda p: (p, 0, 0)),
            pl.BlockSpec((1, d, n), lambda p: (p, 0, 0)),
        ],
        out_specs=[
            pl.BlockSpec((1, 1, 128), lambda p: (p, 0, 0)),
            pl.BlockSpec((1, 1, 128), lambda p: (p, 0, 0)),
        ],
        out_shape=[
            jax.ShapeDtypeStruct((b, 1, 128), jnp.float32),
            jax.ShapeDtypeStruct((b, 1, 128), jnp.float32),
        ],
        scratch_shapes=[
            pltpu.VMEM((n, 1), jnp.float32),
            pltpu.VMEM((1, n), jnp.float32),
            pltpu.VMEM((n, 1), jnp.float32),
            pltpu.VMEM((1, n), jnp.float32),
            pltpu.VMEM((n, d), jnp.float32),
            pltpu.VMEM((d, n), jnp.float32),
        ],
        compiler_params=pltpu.CompilerParams(
            dimension_semantics=("parallel",)),
    )(eps_arr, xs, yts)


def _argmin_body(x_ref, gt_ref, idx_ref, g2_ref):
    g2_ref[...] = jnp.sum(gt_ref[...] * gt_ref[...], axis=0, keepdims=True)

    xb = -2.0 * x_ref[...]
    nc = gt_ref.shape[1] // _BLK

    def chunk(jc, carry):
        m, idx = carry
        j0 = jc * _BLK
        gtc = gt_ref[:, pl.ds(j0, _BLK)]
        g2c = g2_ref[:, pl.ds(j0, _BLK)]
        xy = lax.dot_general(xb, gtc, (((1,), (0,)), ((), ())),
                             preferred_element_type=jnp.float32)
        d = g2c + xy
        cm = jnp.min(d, axis=1, keepdims=True)
        li = lax.broadcasted_iota(jnp.int32, d.shape, 1) + j0
        cidx = jnp.min(jnp.where(d == cm, li, jnp.int32(2 ** 30)), axis=1,
                       keepdims=True)
        upd = cm < m
        return jnp.where(upd, cm, m), jnp.where(upd, cidx, idx)

    carry = (jnp.full((xb.shape[0], 1), 3e38, jnp.float32),
             jnp.zeros((xb.shape[0], 1), jnp.int32))
    for jc in range(nc):
        carry = chunk(jc, carry)
    idx_ref[...] = carry[1]


def _argmin(p8, gt_t):
    n, d = p8.shape
    l = gt_t.shape[1]
    return pl.pallas_call(
        _argmin_body,
        grid=(n // _BLK,),
        in_specs=[
            pl.BlockSpec((_BLK, d), lambda i: (i, 0)),
            pl.BlockSpec((d, l), lambda i: (0, 0)),
        ],
        out_specs=pl.BlockSpec((_BLK, 1), lambda i: (i, 0)),
        out_shape=jax.ShapeDtypeStruct((n, 1), jnp.int32),
        scratch_shapes=[pltpu.VMEM((1, l), jnp.float32)],
        compiler_params=pltpu.CompilerParams(
            dimension_semantics=("parallel",)),
    )(p8, gt_t)


def _moments_body(p_ref, g_ref, cov_ref, mu_ref, vs_ref):
    p = p_ref[...]
    g = g_ref[...]
    n = p.shape[0]
    mu_s = jnp.sum(p, axis=0, keepdims=True) / n
    mu_d = jnp.sum(g, axis=0, keepdims=True) / g.shape[0]
    sc = p - mu_s
    dc = g - mu_d
    cov = lax.dot_general(dc, sc, (((0,), (0,)), ((), ())),
                          preferred_element_type=jnp.float32) / n
    z = jnp.zeros((8, 120), jnp.float32)
    cov_ref[...] = jnp.concatenate([cov, z], axis=1)
    mu8 = jnp.concatenate([mu_s, mu_d, jnp.zeros((6, 8), jnp.float32)],
                          axis=0)
    mu_ref[...] = jnp.concatenate([mu8, z], axis=1)
    vs_ref[...] = (jnp.sum(sc * sc) / n) * jnp.ones((8, 128), jnp.float32)


def _moments(p8, g8):
    return pl.pallas_call(
        _moments_body,
        out_shape=[
            jax.ShapeDtypeStruct((8, 128), jnp.float32),
            jax.ShapeDtypeStruct((8, 128), jnp.float32),
            jax.ShapeDtypeStruct((8, 128), jnp.float32),
        ],
    )(p8, g8)


def _sc_gather(table, idx):
    info = plsc.get_sparse_core_info()
    nc, ns = info.num_cores, info.num_subcores
    nw = nc * ns
    b = idx.shape[0]
    d = table.shape[1]
    bpw = b // nw
    mesh = plsc.VectorSubcoreMesh(core_axis_name="c", subcore_axis_name="s")

    @functools.partial(
        pl.kernel, mesh=mesh,
        out_type=jax.ShapeDtypeStruct((b, d), jnp.float32),
        scratch_types=[
            pltpu.VMEM((bpw,), jnp.int32),
            pltpu.VMEM((bpw, d), jnp.float32),
            pltpu.SemaphoreType.DMA,
        ],
    )
    def k(table_hbm, idx_hbm, out_hbm, idx_v, rows_v, sem):
        wid = lax.axis_index("s") * nc + lax.axis_index("c")
        base = wid * bpw
        pltpu.sync_copy(idx_hbm.at[pl.ds(base, bpw)], idx_v)
        pltpu.async_copy(table_hbm.at[idx_v], rows_v, sem).wait()
        pltpu.sync_copy(rows_v, out_hbm.at[pl.ds(base, bpw)])

    return k(table, idx)


def _cos_body(pn_ref, gn_ref, out_ref):
    pn = pn_ref[...]
    gn = gn_ref[:, :8]
    dot = jnp.sum(pn * gn, axis=1, keepdims=True)
    npn = jnp.sqrt(jnp.sum(pn * pn, axis=1, keepdims=True))
    ngn = jnp.sqrt(jnp.sum(gn * gn, axis=1, keepdims=True))
    cos = dot / (jnp.maximum(npn, 1e-12) * jnp.maximum(ngn, 1e-12))
    out_ref[...] = (1.0 - jnp.sum(cos) / pn.shape[0]) * jnp.ones(
        (8, 128), jnp.float32)


def _cos_loss(pn8, gn):
    return pl.pallas_call(
        _cos_body,
        out_shape=jax.ShapeDtypeStruct((8, 128), jnp.float32),
    )(pn8, gn)


def kernel(pred_feat, pred_decoder, input_data, gt_data):
    f32 = jnp.float32
    x8 = jnp.pad(input_data.astype(f32), ((0, 0), (0, 4)))
    y8 = jnp.pad(pred_decoder.astype(f32), ((0, 0), (0, 4)))
    xs = jnp.stack([x8, x8, y8])
    yts = jnp.stack([y8.T, x8.T, y8.T])
    eps_arr = jnp.asarray(_EPS, f32)
    fo, go = _sinkhorn_pairs(xs, yts, eps_arr)
    ot = fo[:, 0, 0] + go[:, 0, 0]
    rec = ot[0] - 0.5 * ot[1] - 0.5 * ot[2]

    p8 = jnp.pad(pred_feat[:, :3].astype(f32), ((0, 0), (0, 5)))
    g8 = jnp.pad(gt_data[:, :3].astype(f32), ((0, 0), (0, 5)))
    idx = _argmin(p8, g8.T)
    cov_o, mu_o, vs_o = _moments(p8, g8)

    table = jnp.pad(gt_data[:, 3:].astype(f32), ((0, 0), (0, 125)))
    gn = _sc_gather(table, idx[:, 0])
    pn8 = jnp.pad(pred_feat[:, 3:].astype(f32), ((0, 0), (0, 5)))
    norm_loss = _cos_loss(pn8, gn)[0, 0]

    c3 = cov_o[:3, :3]
    u, s_vals, vt = jnp.linalg.svd(c3)
    dsign = jnp.sign(jnp.linalg.det(u) * jnp.linalg.det(vt))
    dvec = jnp.array([1.0, 1.0, 0.0], f32) + jnp.array([0.0, 0.0, 1.0],
                                                       f32) * dsign
    r = (u * dvec[None, :]) @ vt
    var_s = vs_o[0, 0]
    scale = jnp.sum(s_vals * dvec) / var_s
    mu_s = mu_o[0, :3]
    mu_d = mu_o[1, :3]
    t = mu_d - scale * (r @ mu_s)
    reg = (jnp.linalg.norm(r - jnp.eye(3, dtype=f32))
           + jnp.linalg.norm(t) + (scale - 1.0) ** 2)

    return _ALPHA * rec + _BETA * reg + _GAMMA * norm_loss


# speedup vs baseline: 1.5638x; 1.5638x over previous
import functools
import math

import jax
import jax.numpy as jnp
from jax import lax
from jax.experimental import pallas as pl
from jax.experimental.pallas import tpu as pltpu
from jax.experimental.pallas import tpu_sc as plsc

_ALPHA, _BETA, _GAMMA = 0.1, 0.45, 0.45
_BLK = 512
_CH = 512
_NEG = -1e30


def _eps_values(blur=0.05, p=2, scaling=0.5, eps0=1.0):
    tgt = blur ** p
    out = []
    e = eps0
    while e > tgt:
        out.append(e)
        e *= scaling
    out += [tgt] * 5
    return out


_EPS = _eps_values()


def _sinkhorn_body(eps_ref, x_ref, yt_ref, of_ref, og_ref,
                   x2_ref, y2_ref, f_ref, g_ref, u1_ref, u2_ref, v_ref):
    n = x_ref.shape[1]
    nb = n // _CH
    log_w = -math.log(n)

    x2_ref[...] = jnp.sum(x_ref[0] * x_ref[0], axis=1, keepdims=True)
    y2_ref[...] = jnp.sum(yt_ref[0] * yt_ref[0], axis=0, keepdims=True)
    f_ref[...] = jnp.zeros_like(f_ref)
    g_ref[...] = jnp.zeros_like(g_ref)

    ones_c = jnp.ones((n, 1), jnp.float32)
    zeros_c = jnp.zeros((n, 2), jnp.float32)
    ones_r = jnp.ones((1, n), jnp.float32)
    zeros_r = jnp.zeros((2, n), jnp.float32)

    log2e = 1.4426950408889634
    ln2 = 0.6931471805599453

    x4 = x_ref[0][:, 0:4]
    half_x2 = 0.5 * x2_ref[...]
    u1_ref[...] = jnp.concatenate([x4, -half_x2, ones_c, zeros_c], axis=1)
    u2_ref[...] = jnp.concatenate([x4, ones_c, -half_x2, zeros_c], axis=1)

    def eps_step(t, _):
        eps = eps_ref[t]
        ie = log2e / eps

        v_ref[...] = jnp.concatenate(
            [yt_ref[0][0:4, :] * ie, ie * ones_r,
             (-0.5 * ie) * y2_ref[...], zeros_r], axis=0)

        def g_block(jb, _):
            j0 = jb * _BLK
            vb = v_ref[:, pl.ds(j0, _BLK)]

            def chunk(ic, carry):
                m, s = carry
                uc = u1_ref[pl.ds(ic * _CH, _CH), :]
                a = lax.dot_general(uc, vb, (((1,), (0,)), ((), ())),
                                    preferred_element_type=jnp.float32)
                mn = jnp.maximum(m, jnp.max(a, axis=0, keepdims=True))
                s = s * jnp.exp2(m - mn) + jnp.sum(jnp.exp2(a - mn), axis=0,
                                                  keepdims=True)
                return mn, s

            carry = (jnp.full((1, _BLK), _NEG, jnp.float32),
                     jnp.zeros((1, _BLK), jnp.float32))
            for ic in range(nb):
                carry = chunk(ic, carry)
            m, s = carry
            g_ref[:, pl.ds(j0, _BLK)] = -eps * (
                (jnp.log2(s) + m) * ln2 + log_w)
            return 0

        lax.fori_loop(0, nb, g_block, 0)

        v_ref[...] = jnp.concatenate(
            [yt_ref[0][0:4, :] * ie,
             (g_ref[...] - 0.5 * y2_ref[...]) * ie, ie * ones_r, zeros_r],
            axis=0)

        def f_block(ib, _):
            i0 = ib * _BLK
            ub = u2_ref[pl.ds(i0, _BLK), :]

            def chunk(jc, carry):
                m, s = carry
                vc = v_ref[:, pl.ds(jc * _CH, _CH)]
                a = lax.dot_general(ub, vc, (((1,), (0,)), ((), ())),
                                    preferred_element_type=jnp.float32)
                mn = jnp.maximum(m, jnp.max(a, axis=1, keepdims=True))
                s = s * jnp.exp2(m - mn) + jnp.sum(jnp.exp2(a - mn), axis=1,
                                                  keepdims=True)
                return mn, s

            carry = (jnp.full((_BLK, 1), _NEG, jnp.float32),
                     jnp.zeros((_BLK, 1), jnp.float32))
            for jc in range(nb):
                carry = chunk(jc, carry)
            m, s = carry
            fnew = -eps * ((jnp.log2(s) + m) * ln2 + log_w)
            f_ref[pl.ds(i0, _BLK), :] = fnew
            u1_ref[pl.ds(i0, _BLK), 4:5] = (
                fnew - 0.5 * x2_ref[pl.ds(i0, _BLK), :])
            return 0

        lax.fori_loop(0, nb, f_block, 0)
        return 0

    lax.fori_loop(0, len(_EPS), eps_step, 0)
    fm = jnp.sum(f_ref[...]) / n
    gm = jnp.sum(g_ref[...]) / n
    of_ref[...] = fm * jnp.ones((1, 1, 128), jnp.float32)
    og_ref[...] = gm * jnp.ones((1, 1, 128), jnp.float32)


def _sinkhorn_pairs(xs, yts, eps_arr):
    b, n, d = xs.shape
    return pl.pallas_call(
        _sinkhorn_body,
        grid=(b,),
        in_specs=[
            pl.BlockSpec(memory_space=pltpu.SMEM),
            pl.BlockSpec((1, n, d), lambda p: (p, 0, 0)),
            pl.BlockSpec((1, d, n), lambda p: (p, 0, 0)),
        ],
        out_specs=[
            pl.BlockSpec((1, 1, 128), lambda p: (p, 0, 0)),
            pl.BlockSpec((1, 1, 128), lambda p: (p, 0, 0)),
        ],
        out_shape=[
            jax.ShapeDtypeStruct((b, 1, 128), jnp.float32),
            jax.ShapeDtypeStruct((b, 1, 128), jnp.float32),
        ],
        scratch_shapes=[
            pltpu.VMEM((n, 1), jnp.float32),
            pltpu.VMEM((1, n), jnp.float32),
            pltpu.VMEM((n, 1), jnp.float32),
            pltpu.VMEM((1, n), jnp.float32),
            pltpu.VMEM((n, d), jnp.float32),
            pltpu.VMEM((n, d), jnp.float32),
            pltpu.VMEM((d, n), jnp.float32),
        ],
        compiler_params=pltpu.CompilerParams(
            dimension_semantics=("parallel",)),
    )(eps_arr, xs, yts)


def _argmin_body(x_ref, gt_ref, idx_ref, g2_ref):
    g2_ref[...] = jnp.sum(gt_ref[...] * gt_ref[...], axis=0, keepdims=True)

    xb = -2.0 * x_ref[...]
    nc = gt_ref.shape[1] // _BLK

    def chunk(jc, carry):
        m, idx = carry
        j0 = jc * _BLK
        gtc = gt_ref[:, pl.ds(j0, _BLK)]
        g2c = g2_ref[:, pl.ds(j0, _BLK)]
        xy = lax.dot_general(xb, gtc, (((1,), (0,)), ((), ())),
                             preferred_element_type=jnp.float32)
        d = g2c + xy
        cm = jnp.min(d, axis=1, keepdims=True)
        li = lax.broadcasted_iota(jnp.int32, d.shape, 1) + j0
        cidx = jnp.min(jnp.where(d == cm, li, jnp.int32(2 ** 30)), axis=1,
                       keepdims=True)
        upd = cm < m
        return jnp.where(upd, cm, m), jnp.where(upd, cidx, idx)

    carry = (jnp.full((xb.shape[0], 1), 3e38, jnp.float32),
             jnp.zeros((xb.shape[0], 1), jnp.int32))
    for jc in range(nc):
        carry = chunk(jc, carry)
    idx_ref[...] = carry[1]


def _argmin(p8, gt_t):
    n, d = p8.shape
    l = gt_t.shape[1]
    return pl.pallas_call(
        _argmin_body,
        grid=(n // _BLK,),
        in_specs=[
            pl.BlockSpec((_BLK, d), lambda i: (i, 0)),
            pl.BlockSpec((d, l), lambda i: (0, 0)),
        ],
        out_specs=pl.BlockSpec((_BLK, 1), lambda i: (i, 0)),
        out_shape=jax.ShapeDtypeStruct((n, 1), jnp.int32),
        scratch_shapes=[pltpu.VMEM((1, l), jnp.float32)],
        compiler_params=pltpu.CompilerParams(
            dimension_semantics=("parallel",)),
    )(p8, gt_t)


def _moments_body(p_ref, g_ref, cov_ref, mu_ref, vs_ref):
    p = p_ref[...]
    g = g_ref[...]
    n = p.shape[0]
    mu_s = jnp.sum(p, axis=0, keepdims=True) / n
    mu_d = jnp.sum(g, axis=0, keepdims=True) / g.shape[0]
    sc = p - mu_s
    dc = g - mu_d
    cov = lax.dot_general(dc, sc, (((0,), (0,)), ((), ())),
                          preferred_element_type=jnp.float32) / n
    z = jnp.zeros((8, 120), jnp.float32)
    cov_ref[...] = jnp.concatenate([cov, z], axis=1)
    mu8 = jnp.concatenate([mu_s, mu_d, jnp.zeros((6, 8), jnp.float32)],
                          axis=0)
    mu_ref[...] = jnp.concatenate([mu8, z], axis=1)
    vs_ref[...] = (jnp.sum(sc * sc) / n) * jnp.ones((8, 128), jnp.float32)


def _moments(p8, g8):
    return pl.pallas_call(
        _moments_body,
        out_shape=[
            jax.ShapeDtypeStruct((8, 128), jnp.float32),
            jax.ShapeDtypeStruct((8, 128), jnp.float32),
            jax.ShapeDtypeStruct((8, 128), jnp.float32),
        ],
    )(p8, g8)


def _sc_gather(table, idx):
    info = plsc.get_sparse_core_info()
    nc, ns = info.num_cores, info.num_subcores
    nw = nc * ns
    b = idx.shape[0]
    d = table.shape[1]
    bpw = b // nw
    mesh = plsc.VectorSubcoreMesh(core_axis_name="c", subcore_axis_name="s")

    @functools.partial(
        pl.kernel, mesh=mesh,
        out_type=jax.ShapeDtypeStruct((b, d), jnp.float32),
        scratch_types=[
            pltpu.VMEM((bpw,), jnp.int32),
            pltpu.VMEM((bpw, d), jnp.float32),
            pltpu.SemaphoreType.DMA,
        ],
    )
    def k(table_hbm, idx_hbm, out_hbm, idx_v, rows_v, sem):
        wid = lax.axis_index("s") * nc + lax.axis_index("c")
        base = wid * bpw
        pltpu.sync_copy(idx_hbm.at[pl.ds(base, bpw)], idx_v)
        pltpu.async_copy(table_hbm.at[idx_v], rows_v, sem).wait()
        pltpu.sync_copy(rows_v, out_hbm.at[pl.ds(base, bpw)])

    return k(table, idx)


def _cos_body(pn_ref, gn_ref, out_ref):
    pn = pn_ref[...]
    gn = gn_ref[:, :8]
    dot = jnp.sum(pn * gn, axis=1, keepdims=True)
    npn = jnp.sqrt(jnp.sum(pn * pn, axis=1, keepdims=True))
    ngn = jnp.sqrt(jnp.sum(gn * gn, axis=1, keepdims=True))
    cos = dot / (jnp.maximum(npn, 1e-12) * jnp.maximum(ngn, 1e-12))
    out_ref[...] = (1.0 - jnp.sum(cos) / pn.shape[0]) * jnp.ones(
        (8, 128), jnp.float32)


def _cos_loss(pn8, gn):
    return pl.pallas_call(
        _cos_body,
        out_shape=jax.ShapeDtypeStruct((8, 128), jnp.float32),
    )(pn8, gn)


def kernel(pred_feat, pred_decoder, input_data, gt_data):
    f32 = jnp.float32
    x8 = jnp.pad(input_data.astype(f32), ((0, 0), (0, 4)))
    y8 = jnp.pad(pred_decoder.astype(f32), ((0, 0), (0, 4)))
    xs = jnp.stack([x8, x8, y8])
    yts = jnp.stack([y8.T, x8.T, y8.T])
    eps_arr = jnp.asarray(_EPS, f32)
    fo, go = _sinkhorn_pairs(xs, yts, eps_arr)
    ot = fo[:, 0, 0] + go[:, 0, 0]
    rec = ot[0] - 0.5 * ot[1] - 0.5 * ot[2]

    p8 = jnp.pad(pred_feat[:, :3].astype(f32), ((0, 0), (0, 5)))
    g8 = jnp.pad(gt_data[:, :3].astype(f32), ((0, 0), (0, 5)))
    idx = _argmin(p8, g8.T)
    cov_o, mu_o, vs_o = _moments(p8, g8)

    table = jnp.pad(gt_data[:, 3:].astype(f32), ((0, 0), (0, 125)))
    gn = _sc_gather(table, idx[:, 0])
    pn8 = jnp.pad(pred_feat[:, 3:].astype(f32), ((0, 0), (0, 5)))
    norm_loss = _cos_loss(pn8, gn)[0, 0]

    c3 = cov_o[:3, :3]
    u, s_vals, vt = jnp.linalg.svd(c3)
    dsign = jnp.sign(jnp.linalg.det(u) * jnp.linalg.det(vt))
    dvec = jnp.array([1.0, 1.0, 0.0], f32) + jnp.array([0.0, 0.0, 1.0],
                                                       f32) * dsign
    r = (u * dvec[None, :]) @ vt
    var_s = vs_o[0, 0]
    scale = jnp.sum(s_vals * dvec) / var_s
    mu_s = mu_o[0, :3]
    mu_d = mu_o[1, :3]
    t = mu_d - scale * (r @ mu_s)
    reg = (jnp.linalg.norm(r - jnp.eye(3, dtype=f32))
           + jnp.linalg.norm(t) + (scale - 1.0) ** 2)

    return _ALPHA * rec + _BETA * reg + _GAMMA * norm_loss


# speedup vs baseline: 1.6164x; 1.0336x over previous
import functools
import math

import jax
import jax.numpy as jnp
from jax import lax
from jax.experimental import pallas as pl
from jax.experimental.pallas import tpu as pltpu
from jax.experimental.pallas import tpu_sc as plsc

_ALPHA, _BETA, _GAMMA = 0.1, 0.45, 0.45
_BLK = 512
_CH = 512
_NEG = -1e30


def _eps_values(blur=0.05, p=2, scaling=0.5, eps0=1.0):
    tgt = blur ** p
    out = []
    e = eps0
    while e > tgt:
        out.append(e)
        e *= scaling
    out += [tgt] * 5
    return out


_EPS = _eps_values()


def _sinkhorn_body(eps_ref, x_ref, yt_ref, of_ref, og_ref,
                   x2_ref, y2_ref, f_ref, g_ref, u1_ref, u2_ref, v_ref):
    n = x_ref.shape[1]
    nb = n // _CH
    log_w = -math.log(n)

    x2_ref[...] = jnp.sum(x_ref[0] * x_ref[0], axis=1, keepdims=True)
    y2_ref[...] = jnp.sum(yt_ref[0] * yt_ref[0], axis=0, keepdims=True)
    f_ref[...] = jnp.zeros_like(f_ref)
    g_ref[...] = jnp.zeros_like(g_ref)

    ones_c = jnp.ones((n, 1), jnp.float32)
    zeros_c = jnp.zeros((n, 2), jnp.float32)
    ones_r = jnp.ones((1, n), jnp.float32)
    zeros_r = jnp.zeros((2, n), jnp.float32)

    log2e = 1.4426950408889634
    ln2 = 0.6931471805599453

    x4 = x_ref[0][:, 0:4]
    half_x2 = 0.5 * x2_ref[...]
    u2_ref[...] = jnp.concatenate([x4, ones_c, -half_x2, zeros_c], axis=1)

    def eps_step(t, _):
        eps = eps_ref[t]
        ie = log2e / eps

        u1_ref[...] = jnp.concatenate(
            [x4, f_ref[...] - half_x2, ones_c, zeros_c], axis=1)
        v_ref[...] = jnp.concatenate(
            [yt_ref[0][0:4, :] * ie, ie * ones_r,
             (-0.5 * ie) * y2_ref[...], zeros_r], axis=0)

        def g_block(jb, _):
            j0 = jb * _BLK
            vb = v_ref[:, pl.ds(j0, _BLK)]

            def chunk(ic, carry):
                m, s = carry
                uc = u1_ref[pl.ds(ic * _CH, _CH), :]
                a = lax.dot_general(uc, vb, (((1,), (0,)), ((), ())),
                                    preferred_element_type=jnp.float32)
                mn = jnp.maximum(m, jnp.max(a, axis=0, keepdims=True))
                s = s * jnp.exp2(m - mn) + jnp.sum(jnp.exp2(a - mn), axis=0,
                                                  keepdims=True)
                return mn, s

            carry = (jnp.full((1, _BLK), _NEG, jnp.float32),
                     jnp.zeros((1, _BLK), jnp.float32))
            for ic in range(nb):
                carry = chunk(ic, carry)
            m, s = carry
            g_ref[:, pl.ds(j0, _BLK)] = -eps * (
                (jnp.log2(s) + m) * ln2 + log_w)
            return 0

        lax.fori_loop(0, nb, g_block, 0)

        v_ref[...] = jnp.concatenate(
            [yt_ref[0][0:4, :] * ie,
             (g_ref[...] - 0.5 * y2_ref[...]) * ie, ie * ones_r, zeros_r],
            axis=0)

        def f_block(ib, _):
            i0 = ib * _BLK
            ub = u2_ref[pl.ds(i0, _BLK), :]

            def chunk(jc, carry):
                m, s = carry
                vc = v_ref[:, pl.ds(jc * _CH, _CH)]
                a = lax.dot_general(ub, vc, (((1,), (0,)), ((), ())),
                                    preferred_element_type=jnp.float32)
                mn = jnp.maximum(m, jnp.max(a, axis=1, keepdims=True))
                s = s * jnp.exp2(m - mn) + jnp.sum(jnp.exp2(a - mn), axis=1,
                                                  keepdims=True)
                return mn, s

            carry = (jnp.full((_BLK, 1), _NEG, jnp.float32),
                     jnp.zeros((_BLK, 1), jnp.float32))
            for jc in range(nb):
                carry = chunk(jc, carry)
            m, s = carry
            f_ref[pl.ds(i0, _BLK), :] = -eps * (
                (jnp.log2(s) + m) * ln2 + log_w)
            return 0

        lax.fori_loop(0, nb, f_block, 0)
        return 0

    lax.fori_loop(0, len(_EPS), eps_step, 0)
    fm = jnp.sum(f_ref[...]) / n
    gm = jnp.sum(g_ref[...]) / n
    of_ref[...] = fm * jnp.ones((1, 1, 128), jnp.float32)
    og_ref[...] = gm * jnp.ones((1, 1, 128), jnp.float32)


def _sinkhorn_pairs(xs, yts, eps_arr):
    b, n, d = xs.shape
    return pl.pallas_call(
        _sinkhorn_body,
        grid=(b,),
        in_specs=[
            pl.BlockSpec(memory_space=pltpu.SMEM),
            pl.BlockSpec((1, n, d), lambda p: (p, 0, 0)),
            pl.BlockSpec((1, d, n), lambda p: (p, 0, 0)),
        ],
        out_specs=[
            pl.BlockSpec((1, 1, 128), lambda p: (p, 0, 0)),
            pl.BlockSpec((1, 1, 128), lambda p: (p, 0, 0)),
        ],
        out_shape=[
            jax.ShapeDtypeStruct((b, 1, 128), jnp.float32),
            jax.ShapeDtypeStruct((b, 1, 128), jnp.float32),
        ],
        scratch_shapes=[
            pltpu.VMEM((n, 1), jnp.float32),
            pltpu.VMEM((1, n), jnp.float32),
            pltpu.VMEM((n, 1), jnp.float32),
            pltpu.VMEM((1, n), jnp.float32),
            pltpu.VMEM((n, d), jnp.float32),
            pltpu.VMEM((n, d), jnp.float32),
            pltpu.VMEM((d, n), jnp.float32),
        ],
        compiler_params=pltpu.CompilerParams(
            dimension_semantics=("parallel",)),
    )(eps_arr, xs, yts)


def _argmin_body(x_ref, gt_ref, idx_ref, g2_ref):
    g2_ref[...] = jnp.sum(gt_ref[...] * gt_ref[...], axis=0, keepdims=True)

    xb = -2.0 * x_ref[...]
    nc = gt_ref.shape[1] // _BLK

    def chunk(jc, carry):
        m, idx = carry
        j0 = jc * _BLK
        gtc = gt_ref[:, pl.ds(j0, _BLK)]
        g2c = g2_ref[:, pl.ds(j0, _BLK)]
        xy = lax.dot_general(xb, gtc, (((1,), (0,)), ((), ())),
                             preferred_element_type=jnp.float32)
        d = g2c + xy
        cm = jnp.min(d, axis=1, keepdims=True)
        li = lax.broadcasted_iota(jnp.int32, d.shape, 1) + j0
        cidx = jnp.min(jnp.where(d == cm, li, jnp.int32(2 ** 30)), axis=1,
                       keepdims=True)
        upd = cm < m
        return jnp.where(upd, cm, m), jnp.where(upd, cidx, idx)

    carry = (jnp.full((xb.shape[0], 1), 3e38, jnp.float32),
             jnp.zeros((xb.shape[0], 1), jnp.int32))
    for jc in range(nc):
        carry = chunk(jc, carry)
    idx_ref[...] = carry[1]


def _argmin(p8, gt_t):
    n, d = p8.shape
    l = gt_t.shape[1]
    return pl.pallas_call(
        _argmin_body,
        grid=(n // _BLK,),
        in_specs=[
            pl.BlockSpec((_BLK, d), lambda i: (i, 0)),
            pl.BlockSpec((d, l), lambda i: (0, 0)),
        ],
        out_specs=pl.BlockSpec((_BLK, 1), lambda i: (i, 0)),
        out_shape=jax.ShapeDtypeStruct((n, 1), jnp.int32),
        scratch_shapes=[pltpu.VMEM((1, l), jnp.float32)],
        compiler_params=pltpu.CompilerParams(
            dimension_semantics=("parallel",)),
    )(p8, gt_t)


def _moments_body(p_ref, g_ref, cov_ref, mu_ref, vs_ref):
    p = p_ref[...]
    g = g_ref[...]
    n = p.shape[0]
    mu_s = jnp.sum(p, axis=0, keepdims=True) / n
    mu_d = jnp.sum(g, axis=0, keepdims=True) / g.shape[0]
    sc = p - mu_s
    dc = g - mu_d
    cov = lax.dot_general(dc, sc, (((0,), (0,)), ((), ())),
                          preferred_element_type=jnp.float32) / n
    z = jnp.zeros((8, 120), jnp.float32)
    cov_ref[...] = jnp.concatenate([cov, z], axis=1)
    mu8 = jnp.concatenate([mu_s, mu_d, jnp.zeros((6, 8), jnp.float32)],
                          axis=0)
    mu_ref[...] = jnp.concatenate([mu8, z], axis=1)
    vs_ref[...] = (jnp.sum(sc * sc) / n) * jnp.ones((8, 128), jnp.float32)


def _moments(p8, g8):
    return pl.pallas_call(
        _moments_body,
        out_shape=[
            jax.ShapeDtypeStruct((8, 128), jnp.float32),
            jax.ShapeDtypeStruct((8, 128), jnp.float32),
            jax.ShapeDtypeStruct((8, 128), jnp.float32),
        ],
    )(p8, g8)


def _sc_gather(table, idx):
    info = plsc.get_sparse_core_info()
    nc, ns = info.num_cores, info.num_subcores
    nw = nc * ns
    b = idx.shape[0]
    d = table.shape[1]
    bpw = b // nw
    mesh = plsc.VectorSubcoreMesh(core_axis_name="c", subcore_axis_name="s")

    @functools.partial(
        pl.kernel, mesh=mesh,
        out_type=jax.ShapeDtypeStruct((b, d), jnp.float32),
        scratch_types=[
            pltpu.VMEM((bpw,), jnp.int32),
            pltpu.VMEM((bpw, d), jnp.float32),
            pltpu.SemaphoreType.DMA,
        ],
    )
    def k(table_hbm, idx_hbm, out_hbm, idx_v, rows_v, sem):
        wid = lax.axis_index("s") * nc + lax.axis_index("c")
        base = wid * bpw
        pltpu.sync_copy(idx_hbm.at[pl.ds(base, bpw)], idx_v)
        pltpu.async_copy(table_hbm.at[idx_v], rows_v, sem).wait()
        pltpu.sync_copy(rows_v, out_hbm.at[pl.ds(base, bpw)])

    return k(table, idx)


def _cos_body(pn_ref, gn_ref, out_ref):
    pn = pn_ref[...]
    gn = gn_ref[:, :8]
    dot = jnp.sum(pn * gn, axis=1, keepdims=True)
    npn = jnp.sqrt(jnp.sum(pn * pn, axis=1, keepdims=True))
    ngn = jnp.sqrt(jnp.sum(gn * gn, axis=1, keepdims=True))
    cos = dot / (jnp.maximum(npn, 1e-12) * jnp.maximum(ngn, 1e-12))
    out_ref[...] = (1.0 - jnp.sum(cos) / pn.shape[0]) * jnp.ones(
        (8, 128), jnp.float32)


def _cos_loss(pn8, gn):
    return pl.pallas_call(
        _cos_body,
        out_shape=jax.ShapeDtypeStruct((8, 128), jnp.float32),
    )(pn8, gn)


def kernel(pred_feat, pred_decoder, input_data, gt_data):
    f32 = jnp.float32
    x8 = jnp.pad(input_data.astype(f32), ((0, 0), (0, 4)))
    y8 = jnp.pad(pred_decoder.astype(f32), ((0, 0), (0, 4)))
    xs = jnp.stack([x8, x8, y8])
    yts = jnp.stack([y8.T, x8.T, y8.T])
    eps_arr = jnp.asarray(_EPS, f32)
    fo, go = _sinkhorn_pairs(xs, yts, eps_arr)
    ot = fo[:, 0, 0] + go[:, 0, 0]
    rec = ot[0] - 0.5 * ot[1] - 0.5 * ot[2]

    p8 = jnp.pad(pred_feat[:, :3].astype(f32), ((0, 0), (0, 5)))
    g8 = jnp.pad(gt_data[:, :3].astype(f32), ((0, 0), (0, 5)))
    idx = _argmin(p8, g8.T)
    cov_o, mu_o, vs_o = _moments(p8, g8)

    table = jnp.pad(gt_data[:, 3:].astype(f32), ((0, 0), (0, 125)))
    gn = _sc_gather(table, idx[:, 0])
    pn8 = jnp.pad(pred_feat[:, 3:].astype(f32), ((0, 0), (0, 5)))
    norm_loss = _cos_loss(pn8, gn)[0, 0]

    c3 = cov_o[:3, :3]
    u, s_vals, vt = jnp.linalg.svd(c3)
    dsign = jnp.sign(jnp.linalg.det(u) * jnp.linalg.det(vt))
    dvec = jnp.array([1.0, 1.0, 0.0], f32) + jnp.array([0.0, 0.0, 1.0],
                                                       f32) * dsign
    r = (u * dvec[None, :]) @ vt
    var_s = vs_o[0, 0]
    scale = jnp.sum(s_vals * dvec) / var_s
    mu_s = mu_o[0, :3]
    mu_d = mu_o[1, :3]
    t = mu_d - scale * (r @ mu_s)
    reg = (jnp.linalg.norm(r - jnp.eye(3, dtype=f32))
           + jnp.linalg.norm(t) + (scale - 1.0) ** 2)

    return _ALPHA * rec + _BETA * reg + _GAMMA * norm_loss


# speedup vs baseline: 1.6848x; 1.0423x over previous
import functools
import math

import jax
import jax.numpy as jnp
from jax import lax
from jax.experimental import pallas as pl
from jax.experimental.pallas import tpu as pltpu
from jax.experimental.pallas import tpu_sc as plsc

_ALPHA, _BETA, _GAMMA = 0.1, 0.45, 0.45
_BLK = 512
_CH = 512
_NEG = -1e30


def _eps_values(blur=0.05, p=2, scaling=0.5, eps0=1.0):
    tgt = blur ** p
    out = []
    e = eps0
    while e > tgt:
        out.append(e)
        e *= scaling
    out += [tgt] * 5
    return out


_EPS = _eps_values()


def _sinkhorn_body(eps_ref, x_ref, yt_ref, of_ref, og_ref,
                   x2_ref, y2_ref, f_ref, g_ref, u1_ref, u2_ref, v_ref):
    n = x_ref.shape[1]
    nb = n // _CH
    log_w = -math.log(n)

    x2_ref[...] = jnp.sum(x_ref[0] * x_ref[0], axis=1, keepdims=True)
    y2_ref[...] = jnp.sum(yt_ref[0] * yt_ref[0], axis=0, keepdims=True)
    f_ref[...] = jnp.zeros_like(f_ref)
    g_ref[...] = jnp.zeros_like(g_ref)

    ones_c = jnp.ones((n, 1), jnp.float32)
    zeros_c = jnp.zeros((n, 2), jnp.float32)
    ones_r = jnp.ones((1, n), jnp.float32)
    zeros_r = jnp.zeros((2, n), jnp.float32)

    log2e = 1.4426950408889634
    ln2 = 0.6931471805599453

    x4 = x_ref[0][:, 0:4]
    half_x2 = 0.5 * x2_ref[...]
    u2_ref[...] = jnp.concatenate([x4, ones_c, -half_x2, zeros_c], axis=1)

    def eps_step(t, _):
        eps = eps_ref[t]
        ie = log2e / eps

        u1_ref[...] = jnp.concatenate(
            [x4, f_ref[...] - half_x2, ones_c, zeros_c], axis=1)
        v_ref[...] = jnp.concatenate(
            [yt_ref[0][0:4, :] * ie, ie * ones_r,
             (-0.5 * ie) * y2_ref[...], zeros_r], axis=0)

        def g_block(jb, _):
            j0 = jb * _BLK
            vb = v_ref[:, pl.ds(j0, _BLK)]

            def chunk(ic, carry):
                m, s = carry
                uc = u1_ref[pl.ds(ic * _CH, _CH), :]
                a = lax.dot_general(uc, vb, (((1,), (0,)), ((), ())),
                                    preferred_element_type=jnp.float32)
                mn = jnp.maximum(m, jnp.max(a, axis=0, keepdims=True))
                s = s * jnp.exp2(m - mn) + jnp.sum(jnp.exp2(a - mn), axis=0,
                                                  keepdims=True)
                return mn, s

            carry = (jnp.full((1, _BLK), _NEG, jnp.float32),
                     jnp.zeros((1, _BLK), jnp.float32))
            for ic in range(nb):
                carry = chunk(ic, carry)
            m, s = carry
            g_ref[:, pl.ds(j0, _BLK)] = -eps * (
                (jnp.log2(s) + m) * ln2 + log_w)
            return 0

        lax.fori_loop(0, nb, g_block, 0, unroll=2)

        v_ref[...] = jnp.concatenate(
            [yt_ref[0][0:4, :] * ie,
             (g_ref[...] - 0.5 * y2_ref[...]) * ie, ie * ones_r, zeros_r],
            axis=0)

        def f_block(ib, _):
            i0 = ib * _BLK
            ub = u2_ref[pl.ds(i0, _BLK), :]

            def chunk(jc, carry):
                m, s = carry
                vc = v_ref[:, pl.ds(jc * _CH, _CH)]
                a = lax.dot_general(ub, vc, (((1,), (0,)), ((), ())),
                                    preferred_element_type=jnp.float32)
                mn = jnp.maximum(m, jnp.max(a, axis=1, keepdims=True))
                s = s * jnp.exp2(m - mn) + jnp.sum(jnp.exp2(a - mn), axis=1,
                                                  keepdims=True)
                return mn, s

            carry = (jnp.full((_BLK, 1), _NEG, jnp.float32),
                     jnp.zeros((_BLK, 1), jnp.float32))
            for jc in range(nb):
                carry = chunk(jc, carry)
            m, s = carry
            f_ref[pl.ds(i0, _BLK), :] = -eps * (
                (jnp.log2(s) + m) * ln2 + log_w)
            return 0

        lax.fori_loop(0, nb, f_block, 0, unroll=2)
        return 0

    lax.fori_loop(0, len(_EPS), eps_step, 0)
    fm = jnp.sum(f_ref[...]) / n
    gm = jnp.sum(g_ref[...]) / n
    of_ref[...] = fm * jnp.ones((1, 1, 128), jnp.float32)
    og_ref[...] = gm * jnp.ones((1, 1, 128), jnp.float32)


def _sinkhorn_pairs(xs, yts, eps_arr):
    b, n, d = xs.shape
    return pl.pallas_call(
        _sinkhorn_body,
        grid=(b,),
        in_specs=[
            pl.BlockSpec(memory_space=pltpu.SMEM),
            pl.BlockSpec((1, n, d), lambda p: (p, 0, 0)),
            pl.BlockSpec((1, d, n), lambda p: (p, 0, 0)),
        ],
        out_specs=[
            pl.BlockSpec((1, 1, 128), lambda p: (p, 0, 0)),
            pl.BlockSpec((1, 1, 128), lambda p: (p, 0, 0)),
        ],
        out_shape=[
            jax.ShapeDtypeStruct((b, 1, 128), jnp.float32),
            jax.ShapeDtypeStruct((b, 1, 128), jnp.float32),
        ],
        scratch_shapes=[
            pltpu.VMEM((n, 1), jnp.float32),
            pltpu.VMEM((1, n), jnp.float32),
            pltpu.VMEM((n, 1), jnp.float32),
            pltpu.VMEM((1, n), jnp.float32),
            pltpu.VMEM((n, d), jnp.float32),
            pltpu.VMEM((n, d), jnp.float32),
            pltpu.VMEM((d, n), jnp.float32),
        ],
        compiler_params=pltpu.CompilerParams(
            dimension_semantics=("parallel",)),
    )(eps_arr, xs, yts)


def _argmin_body(x_ref, gt_ref, idx_ref, g2_ref):
    g2_ref[...] = jnp.sum(gt_ref[...] * gt_ref[...], axis=0, keepdims=True)

    xb = -2.0 * x_ref[...]
    nc = gt_ref.shape[1] // _BLK

    def chunk(jc, carry):
        m, idx = carry
        j0 = jc * _BLK
        gtc = gt_ref[:, pl.ds(j0, _BLK)]
        g2c = g2_ref[:, pl.ds(j0, _BLK)]
        xy = lax.dot_general(xb, gtc, (((1,), (0,)), ((), ())),
                             preferred_element_type=jnp.float32)
        d = g2c + xy
        cm = jnp.min(d, axis=1, keepdims=True)
        li = lax.broadcasted_iota(jnp.int32, d.shape, 1) + j0
        cidx = jnp.min(jnp.where(d == cm, li, jnp.int32(2 ** 30)), axis=1,
                       keepdims=True)
        upd = cm < m
        return jnp.where(upd, cm, m), jnp.where(upd, cidx, idx)

    carry = (jnp.full((xb.shape[0], 1), 3e38, jnp.float32),
             jnp.zeros((xb.shape[0], 1), jnp.int32))
    for jc in range(nc):
        carry = chunk(jc, carry)
    idx_ref[...] = carry[1]


def _argmin(p8, gt_t):
    n, d = p8.shape
    l = gt_t.shape[1]
    return pl.pallas_call(
        _argmin_body,
        grid=(n // _BLK,),
        in_specs=[
            pl.BlockSpec((_BLK, d), lambda i: (i, 0)),
            pl.BlockSpec((d, l), lambda i: (0, 0)),
        ],
        out_specs=pl.BlockSpec((_BLK, 1), lambda i: (i, 0)),
        out_shape=jax.ShapeDtypeStruct((n, 1), jnp.int32),
        scratch_shapes=[pltpu.VMEM((1, l), jnp.float32)],
        compiler_params=pltpu.CompilerParams(
            dimension_semantics=("parallel",)),
    )(p8, gt_t)


def _moments_body(p_ref, g_ref, cov_ref, mu_ref, vs_ref):
    p = p_ref[...]
    g = g_ref[...]
    n = p.shape[0]
    mu_s = jnp.sum(p, axis=0, keepdims=True) / n
    mu_d = jnp.sum(g, axis=0, keepdims=True) / g.shape[0]
    sc = p - mu_s
    dc = g - mu_d
    cov = lax.dot_general(dc, sc, (((0,), (0,)), ((), ())),
                          preferred_element_type=jnp.float32) / n
    z = jnp.zeros((8, 120), jnp.float32)
    cov_ref[...] = jnp.concatenate([cov, z], axis=1)
    mu8 = jnp.concatenate([mu_s, mu_d, jnp.zeros((6, 8), jnp.float32)],
                          axis=0)
    mu_ref[...] = jnp.concatenate([mu8, z], axis=1)
    vs_ref[...] = (jnp.sum(sc * sc) / n) * jnp.ones((8, 128), jnp.float32)


def _moments(p8, g8):
    return pl.pallas_call(
        _moments_body,
        out_shape=[
            jax.ShapeDtypeStruct((8, 128), jnp.float32),
            jax.ShapeDtypeStruct((8, 128), jnp.float32),
            jax.ShapeDtypeStruct((8, 128), jnp.float32),
        ],
    )(p8, g8)


def _sc_gather(table, idx):
    info = plsc.get_sparse_core_info()
    nc, ns = info.num_cores, info.num_subcores
    nw = nc * ns
    b = idx.shape[0]
    d = table.shape[1]
    bpw = b // nw
    mesh = plsc.VectorSubcoreMesh(core_axis_name="c", subcore_axis_name="s")

    @functools.partial(
        pl.kernel, mesh=mesh,
        out_type=jax.ShapeDtypeStruct((b, d), jnp.float32),
        scratch_types=[
            pltpu.VMEM((bpw,), jnp.int32),
            pltpu.VMEM((bpw, d), jnp.float32),
            pltpu.SemaphoreType.DMA,
        ],
    )
    def k(table_hbm, idx_hbm, out_hbm, idx_v, rows_v, sem):
        wid = lax.axis_index("s") * nc + lax.axis_index("c")
        base = wid * bpw
        pltpu.sync_copy(idx_hbm.at[pl.ds(base, bpw)], idx_v)
        pltpu.async_copy(table_hbm.at[idx_v], rows_v, sem).wait()
        pltpu.sync_copy(rows_v, out_hbm.at[pl.ds(base, bpw)])

    return k(table, idx)


def _cos_body(pn_ref, gn_ref, out_ref):
    pn = pn_ref[...]
    gn = gn_ref[:, :8]
    dot = jnp.sum(pn * gn, axis=1, keepdims=True)
    npn = jnp.sqrt(jnp.sum(pn * pn, axis=1, keepdims=True))
    ngn = jnp.sqrt(jnp.sum(gn * gn, axis=1, keepdims=True))
    cos = dot / (jnp.maximum(npn, 1e-12) * jnp.maximum(ngn, 1e-12))
    out_ref[...] = (1.0 - jnp.sum(cos) / pn.shape[0]) * jnp.ones(
        (8, 128), jnp.float32)


def _cos_loss(pn8, gn):
    return pl.pallas_call(
        _cos_body,
        out_shape=jax.ShapeDtypeStruct((8, 128), jnp.float32),
    )(pn8, gn)


def kernel(pred_feat, pred_decoder, input_data, gt_data):
    f32 = jnp.float32
    x8 = jnp.pad(input_data.astype(f32), ((0, 0), (0, 4)))
    y8 = jnp.pad(pred_decoder.astype(f32), ((0, 0), (0, 4)))
    xs = jnp.stack([x8, x8, y8])
    yts = jnp.stack([y8.T, x8.T, y8.T])
    eps_arr = jnp.asarray(_EPS, f32)
    fo, go = _sinkhorn_pairs(xs, yts, eps_arr)
    ot = fo[:, 0, 0] + go[:, 0, 0]
    rec = ot[0] - 0.5 * ot[1] - 0.5 * ot[2]

    p8 = jnp.pad(pred_feat[:, :3].astype(f32), ((0, 0), (0, 5)))
    g8 = jnp.pad(gt_data[:, :3].astype(f32), ((0, 0), (0, 5)))
    idx = _argmin(p8, g8.T)
    cov_o, mu_o, vs_o = _moments(p8, g8)

    table = jnp.pad(gt_data[:, 3:].astype(f32), ((0, 0), (0, 125)))
    gn = _sc_gather(table, idx[:, 0])
    pn8 = jnp.pad(pred_feat[:, 3:].astype(f32), ((0, 0), (0, 5)))
    norm_loss = _cos_loss(pn8, gn)[0, 0]

    c3 = cov_o[:3, :3]
    u, s_vals, vt = jnp.linalg.svd(c3)
    dsign = jnp.sign(jnp.linalg.det(u) * jnp.linalg.det(vt))
    dvec = jnp.array([1.0, 1.0, 0.0], f32) + jnp.array([0.0, 0.0, 1.0],
                                                       f32) * dsign
    r = (u * dvec[None, :]) @ vt
    var_s = vs_o[0, 0]
    scale = jnp.sum(s_vals * dvec) / var_s
    mu_s = mu_o[0, :3]
    mu_d = mu_o[1, :3]
    t = mu_d - scale * (r @ mu_s)
    reg = (jnp.linalg.norm(r - jnp.eye(3, dtype=f32))
           + jnp.linalg.norm(t) + (scale - 1.0) ** 2)

    return _ALPHA * rec + _BETA * reg + _GAMMA * norm_loss


# speedup vs baseline: 1.7236x; 1.0230x over previous
import functools
import math

import jax
import jax.numpy as jnp
from jax import lax
from jax.experimental import pallas as pl
from jax.experimental.pallas import tpu as pltpu
from jax.experimental.pallas import tpu_sc as plsc

_ALPHA, _BETA, _GAMMA = 0.1, 0.45, 0.45
_BLK = 512
_CH = 512
_NEG = -1e30


def _eps_values(blur=0.05, p=2, scaling=0.5, eps0=1.0):
    tgt = blur ** p
    out = []
    e = eps0
    while e > tgt:
        out.append(e)
        e *= scaling
    out += [tgt] * 5
    return out


_EPS = _eps_values()


def _sinkhorn_body(eps_ref, x_ref, yt_ref, of_ref, og_ref,
                   x2_ref, y2_ref, f_ref, g_ref, u1_ref, u2_ref, v_ref):
    n = x_ref.shape[1]
    nb = n // _CH
    log_w = -math.log(n)

    x2_ref[...] = jnp.sum(x_ref[0] * x_ref[0], axis=1, keepdims=True)
    y2_ref[...] = jnp.sum(yt_ref[0] * yt_ref[0], axis=0, keepdims=True)
    f_ref[...] = jnp.zeros_like(f_ref)
    g_ref[...] = jnp.zeros_like(g_ref)

    ones_c = jnp.ones((n, 1), jnp.float32)
    zeros_c = jnp.zeros((n, 2), jnp.float32)
    ones_r = jnp.ones((1, n), jnp.float32)
    zeros_r = jnp.zeros((2, n), jnp.float32)

    log2e = 1.4426950408889634
    ln2 = 0.6931471805599453

    x4 = x_ref[0][:, 0:4]
    half_x2 = 0.5 * x2_ref[...]
    u2_ref[...] = jnp.concatenate([x4, ones_c, -half_x2, zeros_c], axis=1)

    def eps_step(t, _):
        eps = eps_ref[t]
        ie = log2e / eps

        u1_ref[...] = jnp.concatenate(
            [x4, f_ref[...] - half_x2, ones_c, zeros_c], axis=1)
        v_ref[...] = jnp.concatenate(
            [yt_ref[0][0:4, :] * ie, ie * ones_r,
             (-0.5 * ie) * y2_ref[...], zeros_r], axis=0)

        def g_block(jb, _):
            j0 = jb * _BLK
            vb = v_ref[:, pl.ds(j0, _BLK)]

            def chunk(ic, carry):
                m, s = carry
                uc = u1_ref[pl.ds(ic * _CH, _CH), :]
                a = lax.dot_general(uc, vb, (((1,), (0,)), ((), ())),
                                    preferred_element_type=jnp.float32)
                mn = jnp.maximum(m, jnp.max(a, axis=0, keepdims=True))
                s = s * jnp.exp2(m - mn) + jnp.sum(jnp.exp2(a - mn), axis=0,
                                                  keepdims=True)
                return mn, s

            carry = (jnp.full((1, _BLK), _NEG, jnp.float32),
                     jnp.zeros((1, _BLK), jnp.float32))
            for ic in range(nb):
                carry = chunk(ic, carry)
            m, s = carry
            g_ref[:, pl.ds(j0, _BLK)] = -eps * (
                (jnp.log2(s) + m) * ln2 + log_w)
            return 0

        lax.fori_loop(0, nb, g_block, 0, unroll=4)

        v_ref[...] = jnp.concatenate(
            [yt_ref[0][0:4, :] * ie,
             (g_ref[...] - 0.5 * y2_ref[...]) * ie, ie * ones_r, zeros_r],
            axis=0)

        def f_block(ib, _):
            i0 = ib * _BLK
            ub = u2_ref[pl.ds(i0, _BLK), :]

            def chunk(jc, carry):
                m, s = carry
                vc = v_ref[:, pl.ds(jc * _CH, _CH)]
                a = lax.dot_general(ub, vc, (((1,), (0,)), ((), ())),
                                    preferred_element_type=jnp.float32)
                mn = jnp.maximum(m, jnp.max(a, axis=1, keepdims=True))
                s = s * jnp.exp2(m - mn) + jnp.sum(jnp.exp2(a - mn), axis=1,
                                                  keepdims=True)
                return mn, s

            carry = (jnp.full((_BLK, 1), _NEG, jnp.float32),
                     jnp.zeros((_BLK, 1), jnp.float32))
            for jc in range(nb):
                carry = chunk(jc, carry)
            m, s = carry
            f_ref[pl.ds(i0, _BLK), :] = -eps * (
                (jnp.log2(s) + m) * ln2 + log_w)
            return 0

        lax.fori_loop(0, nb, f_block, 0, unroll=4)
        return 0

    lax.fori_loop(0, len(_EPS), eps_step, 0)
    fm = jnp.sum(f_ref[...]) / n
    gm = jnp.sum(g_ref[...]) / n
    of_ref[...] = fm * jnp.ones((1, 1, 128), jnp.float32)
    og_ref[...] = gm * jnp.ones((1, 1, 128), jnp.float32)


def _sinkhorn_pairs(xs, yts, eps_arr):
    b, n, d = xs.shape
    return pl.pallas_call(
        _sinkhorn_body,
        grid=(b,),
        in_specs=[
            pl.BlockSpec(memory_space=pltpu.SMEM),
            pl.BlockSpec((1, n, d), lambda p: (p, 0, 0)),
            pl.BlockSpec((1, d, n), lambda p: (p, 0, 0)),
        ],
        out_specs=[
            pl.BlockSpec((1, 1, 128), lambda p: (p, 0, 0)),
            pl.BlockSpec((1, 1, 128), lambda p: (p, 0, 0)),
        ],
        out_shape=[
            jax.ShapeDtypeStruct((b, 1, 128), jnp.float32),
            jax.ShapeDtypeStruct((b, 1, 128), jnp.float32),
        ],
        scratch_shapes=[
            pltpu.VMEM((n, 1), jnp.float32),
            pltpu.VMEM((1, n), jnp.float32),
            pltpu.VMEM((n, 1), jnp.float32),
            pltpu.VMEM((1, n), jnp.float32),
            pltpu.VMEM((n, d), jnp.float32),
            pltpu.VMEM((n, d), jnp.float32),
            pltpu.VMEM((d, n), jnp.float32),
        ],
        compiler_params=pltpu.CompilerParams(
            dimension_semantics=("parallel",)),
    )(eps_arr, xs, yts)


def _argmin_body(x_ref, gt_ref, idx_ref, g2_ref):
    g2_ref[...] = jnp.sum(gt_ref[...] * gt_ref[...], axis=0, keepdims=True)

    xb = -2.0 * x_ref[...]
    nc = gt_ref.shape[1] // _BLK

    def chunk(jc, carry):
        m, idx = carry
        j0 = jc * _BLK
        gtc = gt_ref[:, pl.ds(j0, _BLK)]
        g2c = g2_ref[:, pl.ds(j0, _BLK)]
        xy = lax.dot_general(xb, gtc, (((1,), (0,)), ((), ())),
                             preferred_element_type=jnp.float32)
        d = g2c + xy
        cm = jnp.min(d, axis=1, keepdims=True)
        li = lax.broadcasted_iota(jnp.int32, d.shape, 1) + j0
        cidx = jnp.min(jnp.where(d == cm, li, jnp.int32(2 ** 30)), axis=1,
                       keepdims=True)
        upd = cm < m
        return jnp.where(upd, cm, m), jnp.where(upd, cidx, idx)

    carry = (jnp.full((xb.shape[0], 1), 3e38, jnp.float32),
             jnp.zeros((xb.shape[0], 1), jnp.int32))
    for jc in range(nc):
        carry = chunk(jc, carry)
    idx_ref[...] = carry[1]


def _argmin(p8, gt_t):
    n, d = p8.shape
    l = gt_t.shape[1]
    return pl.pallas_call(
        _argmin_body,
        grid=(n // _BLK,),
        in_specs=[
            pl.BlockSpec((_BLK, d), lambda i: (i, 0)),
            pl.BlockSpec((d, l), lambda i: (0, 0)),
        ],
        out_specs=pl.BlockSpec((_BLK, 1), lambda i: (i, 0)),
        out_shape=jax.ShapeDtypeStruct((n, 1), jnp.int32),
        scratch_shapes=[pltpu.VMEM((1, l), jnp.float32)],
        compiler_params=pltpu.CompilerParams(
            dimension_semantics=("parallel",)),
    )(p8, gt_t)


def _moments_body(p_ref, g_ref, cov_ref, mu_ref, vs_ref):
    p = p_ref[...]
    g = g_ref[...]
    n = p.shape[0]
    mu_s = jnp.sum(p, axis=0, keepdims=True) / n
    mu_d = jnp.sum(g, axis=0, keepdims=True) / g.shape[0]
    sc = p - mu_s
    dc = g - mu_d
    cov = lax.dot_general(dc, sc, (((0,), (0,)), ((), ())),
                          preferred_element_type=jnp.float32) / n
    z = jnp.zeros((8, 120), jnp.float32)
    cov_ref[...] = jnp.concatenate([cov, z], axis=1)
    mu8 = jnp.concatenate([mu_s, mu_d, jnp.zeros((6, 8), jnp.float32)],
                          axis=0)
    mu_ref[...] = jnp.concatenate([mu8, z], axis=1)
    vs_ref[...] = (jnp.sum(sc * sc) / n) * jnp.ones((8, 128), jnp.float32)


def _moments(p8, g8):
    return pl.pallas_call(
        _moments_body,
        out_shape=[
            jax.ShapeDtypeStruct((8, 128), jnp.float32),
            jax.ShapeDtypeStruct((8, 128), jnp.float32),
            jax.ShapeDtypeStruct((8, 128), jnp.float32),
        ],
    )(p8, g8)


def _sc_gather(table, idx):
    info = plsc.get_sparse_core_info()
    nc, ns = info.num_cores, info.num_subcores
    nw = nc * ns
    b = idx.shape[0]
    d = table.shape[1]
    bpw = b // nw
    mesh = plsc.VectorSubcoreMesh(core_axis_name="c", subcore_axis_name="s")

    @functools.partial(
        pl.kernel, mesh=mesh,
        out_type=jax.ShapeDtypeStruct((b, d), jnp.float32),
        scratch_types=[
            pltpu.VMEM((bpw,), jnp.int32),
            pltpu.VMEM((bpw, d), jnp.float32),
            pltpu.SemaphoreType.DMA,
        ],
    )
    def k(table_hbm, idx_hbm, out_hbm, idx_v, rows_v, sem):
        wid = lax.axis_index("s") * nc + lax.axis_index("c")
        base = wid * bpw
        pltpu.sync_copy(idx_hbm.at[pl.ds(base, bpw)], idx_v)
        pltpu.async_copy(table_hbm.at[idx_v], rows_v, sem).wait()
        pltpu.sync_copy(rows_v, out_hbm.at[pl.ds(base, bpw)])

    return k(table, idx)


def _cos_body(pn_ref, gn_ref, out_ref):
    pn = pn_ref[...]
    gn = gn_ref[:, :8]
    dot = jnp.sum(pn * gn, axis=1, keepdims=True)
    npn = jnp.sqrt(jnp.sum(pn * pn, axis=1, keepdims=True))
    ngn = jnp.sqrt(jnp.sum(gn * gn, axis=1, keepdims=True))
    cos = dot / (jnp.maximum(npn, 1e-12) * jnp.maximum(ngn, 1e-12))
    out_ref[...] = (1.0 - jnp.sum(cos) / pn.shape[0]) * jnp.ones(
        (8, 128), jnp.float32)


def _cos_loss(pn8, gn):
    return pl.pallas_call(
        _cos_body,
        out_shape=jax.ShapeDtypeStruct((8, 128), jnp.float32),
    )(pn8, gn)


def kernel(pred_feat, pred_decoder, input_data, gt_data):
    f32 = jnp.float32
    x8 = jnp.pad(input_data.astype(f32), ((0, 0), (0, 4)))
    y8 = jnp.pad(pred_decoder.astype(f32), ((0, 0), (0, 4)))
    xs = jnp.stack([x8, x8, y8])
    yts = jnp.stack([y8.T, x8.T, y8.T])
    eps_arr = jnp.asarray(_EPS, f32)
    fo, go = _sinkhorn_pairs(xs, yts, eps_arr)
    ot = fo[:, 0, 0] + go[:, 0, 0]
    rec = ot[0] - 0.5 * ot[1] - 0.5 * ot[2]

    p8 = jnp.pad(pred_feat[:, :3].astype(f32), ((0, 0), (0, 5)))
    g8 = jnp.pad(gt_data[:, :3].astype(f32), ((0, 0), (0, 5)))
    idx = _argmin(p8, g8.T)
    cov_o, mu_o, vs_o = _moments(p8, g8)

    table = jnp.pad(gt_data[:, 3:].astype(f32), ((0, 0), (0, 125)))
    gn = _sc_gather(table, idx[:, 0])
    pn8 = jnp.pad(pred_feat[:, 3:].astype(f32), ((0, 0), (0, 5)))
    norm_loss = _cos_loss(pn8, gn)[0, 0]

    c3 = cov_o[:3, :3]
    u, s_vals, vt = jnp.linalg.svd(c3)
    dsign = jnp.sign(jnp.linalg.det(u) * jnp.linalg.det(vt))
    dvec = jnp.array([1.0, 1.0, 0.0], f32) + jnp.array([0.0, 0.0, 1.0],
                                                       f32) * dsign
    r = (u * dvec[None, :]) @ vt
    var_s = vs_o[0, 0]
    scale = jnp.sum(s_vals * dvec) / var_s
    mu_s = mu_o[0, :3]
    mu_d = mu_o[1, :3]
    t = mu_d - scale * (r @ mu_s)
    reg = (jnp.linalg.norm(r - jnp.eye(3, dtype=f32))
           + jnp.linalg.norm(t) + (scale - 1.0) ** 2)

    return _ALPHA * rec + _BETA * reg + _GAMMA * norm_loss


# speedup vs baseline: 1.8387x; 1.0668x over previous
import functools
import math

import jax
import jax.numpy as jnp
from jax import lax
from jax.experimental import pallas as pl
from jax.experimental.pallas import tpu as pltpu
from jax.experimental.pallas import tpu_sc as plsc

_ALPHA, _BETA, _GAMMA = 0.1, 0.45, 0.45
_BLK = 512
_CH = 512
_NEG = -1e30


def _eps_values(blur=0.05, p=2, scaling=0.5, eps0=1.0):
    tgt = blur ** p
    out = []
    e = eps0
    while e > tgt:
        out.append(e)
        e *= scaling
    out += [tgt] * 5
    return out


_EPS = _eps_values()


def _sinkhorn_body(eps_ref, x_ref, yt_ref, of_ref, og_ref,
                   x2_ref, y2_ref, f_ref, g_ref, u1_ref, u2_ref, v_ref):
    n = x_ref.shape[1]
    nb = n // _CH
    log_w = -math.log(n)

    x2_ref[...] = jnp.sum(x_ref[0] * x_ref[0], axis=1, keepdims=True)
    y2_ref[...] = jnp.sum(yt_ref[0] * yt_ref[0], axis=0, keepdims=True)
    f_ref[...] = jnp.zeros_like(f_ref)
    g_ref[...] = jnp.zeros_like(g_ref)

    ones_c = jnp.ones((n, 1), jnp.float32)
    zeros_c = jnp.zeros((n, 2), jnp.float32)
    ones_r = jnp.ones((1, n), jnp.float32)
    zeros_r = jnp.zeros((2, n), jnp.float32)

    log2e = 1.4426950408889634
    ln2 = 0.6931471805599453

    x4 = x_ref[0][:, 0:4]
    half_x2 = 0.5 * x2_ref[...]
    u2_ref[...] = jnp.concatenate([x4, ones_c, -half_x2, zeros_c], axis=1)

    def eps_step(t, _):
        eps = eps_ref[t]
        ie = log2e / eps

        u1_ref[...] = jnp.concatenate(
            [x4, f_ref[...] - half_x2, ones_c, zeros_c], axis=1)
        v_ref[...] = jnp.concatenate(
            [yt_ref[0][0:4, :] * ie, ie * ones_r,
             (-0.5 * ie) * y2_ref[...], zeros_r], axis=0)

        def g_block(jb, _):
            j0 = jb * _BLK
            vb = v_ref[:, pl.ds(j0, _BLK)]

            def chunk(ic, carry):
                m, s = carry
                uc = u1_ref[pl.ds(ic * _CH, _CH), :]
                a = lax.dot_general(uc, vb, (((1,), (0,)), ((), ())),
                                    preferred_element_type=jnp.float32)
                mn = jnp.maximum(m, jnp.max(a, axis=0, keepdims=True))
                s = s * jnp.exp2(m - mn) + jnp.sum(jnp.exp2(a - mn), axis=0,
                                                  keepdims=True)
                return mn, s

            carry = (jnp.full((1, _BLK), _NEG, jnp.float32),
                     jnp.zeros((1, _BLK), jnp.float32))
            for ic in range(nb):
                carry = chunk(ic, carry)
            m, s = carry
            g_ref[:, pl.ds(j0, _BLK)] = -eps * (
                (jnp.log2(s) + m) * ln2 + log_w)
            return 0

        lax.fori_loop(0, nb, g_block, 0, unroll=8)

        v_ref[...] = jnp.concatenate(
            [yt_ref[0][0:4, :] * ie,
             (g_ref[...] - 0.5 * y2_ref[...]) * ie, ie * ones_r, zeros_r],
            axis=0)

        def f_block(ib, _):
            i0 = ib * _BLK
            ub = u2_ref[pl.ds(i0, _BLK), :]

            def chunk(jc, carry):
                m, s = carry
                vc = v_ref[:, pl.ds(jc * _CH, _CH)]
                a = lax.dot_general(ub, vc, (((1,), (0,)), ((), ())),
                                    preferred_element_type=jnp.float32)
                mn = jnp.maximum(m, jnp.max(a, axis=1, keepdims=True))
                s = s * jnp.exp2(m - mn) + jnp.sum(jnp.exp2(a - mn), axis=1,
                                                  keepdims=True)
                return mn, s

            carry = (jnp.full((_BLK, 1), _NEG, jnp.float32),
                     jnp.zeros((_BLK, 1), jnp.float32))
            for jc in range(nb):
                carry = chunk(jc, carry)
            m, s = carry
            f_ref[pl.ds(i0, _BLK), :] = -eps * (
                (jnp.log2(s) + m) * ln2 + log_w)
            return 0

        lax.fori_loop(0, nb, f_block, 0, unroll=8)
        return 0

    lax.fori_loop(0, len(_EPS), eps_step, 0)
    fm = jnp.sum(f_ref[...]) / n
    gm = jnp.sum(g_ref[...]) / n
    of_ref[...] = fm * jnp.ones((1, 1, 128), jnp.float32)
    og_ref[...] = gm * jnp.ones((1, 1, 128), jnp.float32)


def _sinkhorn_pairs(xs, yts, eps_arr):
    b, n, d = xs.shape
    return pl.pallas_call(
        _sinkhorn_body,
        grid=(b,),
        in_specs=[
            pl.BlockSpec(memory_space=pltpu.SMEM),
            pl.BlockSpec((1, n, d), lambda p: (p, 0, 0)),
            pl.BlockSpec((1, d, n), lambda p: (p, 0, 0)),
        ],
        out_specs=[
            pl.BlockSpec((1, 1, 128), lambda p: (p, 0, 0)),
            pl.BlockSpec((1, 1, 128), lambda p: (p, 0, 0)),
        ],
        out_shape=[
            jax.ShapeDtypeStruct((b, 1, 128), jnp.float32),
            jax.ShapeDtypeStruct((b, 1, 128), jnp.float32),
        ],
        scratch_shapes=[
            pltpu.VMEM((n, 1), jnp.float32),
            pltpu.VMEM((1, n), jnp.float32),
            pltpu.VMEM((n, 1), jnp.float32),
            pltpu.VMEM((1, n), jnp.float32),
            pltpu.VMEM((n, d), jnp.float32),
            pltpu.VMEM((n, d), jnp.float32),
            pltpu.VMEM((d, n), jnp.float32),
        ],
        compiler_params=pltpu.CompilerParams(
            dimension_semantics=("parallel",)),
    )(eps_arr, xs, yts)


def _argmin_body(x_ref, gt_ref, idx_ref, g2_ref):
    g2_ref[...] = jnp.sum(gt_ref[...] * gt_ref[...], axis=0, keepdims=True)

    xb = -2.0 * x_ref[...]
    nc = gt_ref.shape[1] // _BLK

    def chunk(jc, carry):
        m, idx = carry
        j0 = jc * _BLK
        gtc = gt_ref[:, pl.ds(j0, _BLK)]
        g2c = g2_ref[:, pl.ds(j0, _BLK)]
        xy = lax.dot_general(xb, gtc, (((1,), (0,)), ((), ())),
                             preferred_element_type=jnp.float32)
        d = g2c + xy
        cm = jnp.min(d, axis=1, keepdims=True)
        li = lax.broadcasted_iota(jnp.int32, d.shape, 1) + j0
        cidx = jnp.min(jnp.where(d == cm, li, jnp.int32(2 ** 30)), axis=1,
                       keepdims=True)
        upd = cm < m
        return jnp.where(upd, cm, m), jnp.where(upd, cidx, idx)

    carry = (jnp.full((xb.shape[0], 1), 3e38, jnp.float32),
             jnp.zeros((xb.shape[0], 1), jnp.int32))
    for jc in range(nc):
        carry = chunk(jc, carry)
    idx_ref[...] = carry[1]


def _argmin(p8, gt_t):
    n, d = p8.shape
    l = gt_t.shape[1]
    return pl.pallas_call(
        _argmin_body,
        grid=(n // _BLK,),
        in_specs=[
            pl.BlockSpec((_BLK, d), lambda i: (i, 0)),
            pl.BlockSpec((d, l), lambda i: (0, 0)),
        ],
        out_specs=pl.BlockSpec((_BLK, 1), lambda i: (i, 0)),
        out_shape=jax.ShapeDtypeStruct((n, 1), jnp.int32),
        scratch_shapes=[pltpu.VMEM((1, l), jnp.float32)],
        compiler_params=pltpu.CompilerParams(
            dimension_semantics=("parallel",)),
    )(p8, gt_t)


def _moments_body(p_ref, g_ref, cov_ref, mu_ref, vs_ref):
    p = p_ref[...]
    g = g_ref[...]
    n = p.shape[0]
    mu_s = jnp.sum(p, axis=0, keepdims=True) / n
    mu_d = jnp.sum(g, axis=0, keepdims=True) / g.shape[0]
    sc = p - mu_s
    dc = g - mu_d
    cov = lax.dot_general(dc, sc, (((0,), (0,)), ((), ())),
                          preferred_element_type=jnp.float32) / n
    z = jnp.zeros((8, 120), jnp.float32)
    cov_ref[...] = jnp.concatenate([cov, z], axis=1)
    mu8 = jnp.concatenate([mu_s, mu_d, jnp.zeros((6, 8), jnp.float32)],
                          axis=0)
    mu_ref[...] = jnp.concatenate([mu8, z], axis=1)
    vs_ref[...] = (jnp.sum(sc * sc) / n) * jnp.ones((8, 128), jnp.float32)


def _moments(p8, g8):
    return pl.pallas_call(
        _moments_body,
        out_shape=[
            jax.ShapeDtypeStruct((8, 128), jnp.float32),
            jax.ShapeDtypeStruct((8, 128), jnp.float32),
            jax.ShapeDtypeStruct((8, 128), jnp.float32),
        ],
    )(p8, g8)


def _sc_gather(table, idx):
    info = plsc.get_sparse_core_info()
    nc, ns = info.num_cores, info.num_subcores
    nw = nc * ns
    b = idx.shape[0]
    d = table.shape[1]
    bpw = b // nw
    mesh = plsc.VectorSubcoreMesh(core_axis_name="c", subcore_axis_name="s")

    @functools.partial(
        pl.kernel, mesh=mesh,
        out_type=jax.ShapeDtypeStruct((b, d), jnp.float32),
        scratch_types=[
            pltpu.VMEM((bpw,), jnp.int32),
            pltpu.VMEM((bpw, d), jnp.float32),
            pltpu.SemaphoreType.DMA,
        ],
    )
    def k(table_hbm, idx_hbm, out_hbm, idx_v, rows_v, sem):
        wid = lax.axis_index("s") * nc + lax.axis_index("c")
        base = wid * bpw
        pltpu.sync_copy(idx_hbm.at[pl.ds(base, bpw)], idx_v)
        pltpu.async_copy(table_hbm.at[idx_v], rows_v, sem).wait()
        pltpu.sync_copy(rows_v, out_hbm.at[pl.ds(base, bpw)])

    return k(table, idx)


def _cos_body(pn_ref, gn_ref, out_ref):
    pn = pn_ref[...]
    gn = gn_ref[:, :8]
    dot = jnp.sum(pn * gn, axis=1, keepdims=True)
    npn = jnp.sqrt(jnp.sum(pn * pn, axis=1, keepdims=True))
    ngn = jnp.sqrt(jnp.sum(gn * gn, axis=1, keepdims=True))
    cos = dot / (jnp.maximum(npn, 1e-12) * jnp.maximum(ngn, 1e-12))
    out_ref[...] = (1.0 - jnp.sum(cos) / pn.shape[0]) * jnp.ones(
        (8, 128), jnp.float32)


def _cos_loss(pn8, gn):
    return pl.pallas_call(
        _cos_body,
        out_shape=jax.ShapeDtypeStruct((8, 128), jnp.float32),
    )(pn8, gn)


def kernel(pred_feat, pred_decoder, input_data, gt_data):
    f32 = jnp.float32
    x8 = jnp.pad(input_data.astype(f32), ((0, 0), (0, 4)))
    y8 = jnp.pad(pred_decoder.astype(f32), ((0, 0), (0, 4)))
    xs = jnp.stack([x8, x8, y8])
    yts = jnp.stack([y8.T, x8.T, y8.T])
    eps_arr = jnp.asarray(_EPS, f32)
    fo, go = _sinkhorn_pairs(xs, yts, eps_arr)
    ot = fo[:, 0, 0] + go[:, 0, 0]
    rec = ot[0] - 0.5 * ot[1] - 0.5 * ot[2]

    p8 = jnp.pad(pred_feat[:, :3].astype(f32), ((0, 0), (0, 5)))
    g8 = jnp.pad(gt_data[:, :3].astype(f32), ((0, 0), (0, 5)))
    idx = _argmin(p8, g8.T)
    cov_o, mu_o, vs_o = _moments(p8, g8)

    table = jnp.pad(gt_data[:, 3:].astype(f32), ((0, 0), (0, 125)))
    gn = _sc_gather(table, idx[:, 0])
    pn8 = jnp.pad(pred_feat[:, 3:].astype(f32), ((0, 0), (0, 5)))
    norm_loss = _cos_loss(pn8, gn)[0, 0]

    c3 = cov_o[:3, :3]
    u, s_vals, vt = jnp.linalg.svd(c3)
    dsign = jnp.sign(jnp.linalg.det(u) * jnp.linalg.det(vt))
    dvec = jnp.array([1.0, 1.0, 0.0], f32) + jnp.array([0.0, 0.0, 1.0],
                                                       f32) * dsign
    r = (u * dvec[None, :]) @ vt
    var_s = vs_o[0, 0]
    scale = jnp.sum(s_vals * dvec) / var_s
    mu_s = mu_o[0, :3]
    mu_d = mu_o[1, :3]
    t = mu_d - scale * (r @ mu_s)
    reg = (jnp.linalg.norm(r - jnp.eye(3, dtype=f32))
           + jnp.linalg.norm(t) + (scale - 1.0) ** 2)

    return _ALPHA * rec + _BETA * reg + _GAMMA * norm_loss
